# baseline (speedup 1.0000x reference)
import jax
import jax.numpy as jnp
from jax.experimental import pallas as pl
from jax.experimental.pallas import tpu as pltpu

_TOPK = 200
_NUM_PERS = 4
_SUB = 40


def _fused_kernel(x_ref, pf_ref, w_ref, out_ref):
    x = x_ref[...]
    pf = pf_ref[...]
    w = w_ref[...]
    acc = jnp.zeros((x.shape[0], pf.shape[0]), jnp.float32)
    for k in range(_NUM_PERS):
        ps = pf * w[k][None, :]
        pn = ps / jnp.maximum(
            jnp.sqrt(jnp.sum(ps * ps, axis=1, keepdims=True)), 1e-8)
        xs = x * w[k][None, :]
        d4 = jnp.maximum(
            jnp.sqrt(jnp.sum(xs * xs, axis=1, keepdims=True)),
            1e-8) * jnp.float32(_NUM_PERS)
        acc = acc + jax.lax.dot_general(
            xs / d4, pn, (((1,), (1,)), ((), ())),
            preferred_element_type=jnp.float32)
    att = acc

    u = jax.lax.bitcast_convert_type(att, jnp.int32)
    key = u ^ ((u >> 31) & jnp.int32(0x7FFFFFFF))

    kf = jnp.float32(_TOPK)
    for t in range(att.shape[0] // _SUB):
        sl = slice(t * _SUB, (t + 1) * _SUB)
        mk = key[sl, :]
        thr = jnp.full((_SUB, 1), jnp.int32(-(1 << 30)))
        for bit in range(30, -1, -1):
            cand = thr + jnp.int32(1 << bit)
            m = jnp.where(mk >= cand, 1.0, 0.0)
            cnt = jnp.sum(m, axis=1, keepdims=True)
            thr = jnp.where(cnt >= kf, cand, thr)
        out_ref[sl, :] = jnp.where(mk >= thr, att[sl, :], 0.0)


def kernel(node_features, pivot_features, weight_tensor):
    n, d = node_features.shape
    p, _ = pivot_features.shape
    block_n = 400
    assert n % block_n == 0
    w_pad = jnp.zeros((8, d), weight_tensor.dtype).at[:_NUM_PERS].set(
        weight_tensor)
    return pl.pallas_call(
        _fused_kernel,
        grid=(n // block_n,),
        in_specs=[
            pl.BlockSpec((block_n, d), lambda i: (i, 0)),
            pl.BlockSpec((p, d), lambda i: (0, 0)),
            pl.BlockSpec((8, d), lambda i: (0, 0)),
        ],
        out_specs=pl.BlockSpec((block_n, p), lambda i: (i, 0)),
        out_shape=jax.ShapeDtypeStruct((n, p), jnp.float32),
        compiler_params=pltpu.CompilerParams(
            dimension_semantics=("parallel",)),
    )(node_features, pivot_features, w_pad)


# speedup vs baseline: 1.0768x; 1.0768x over previous
import jax
import jax.numpy as jnp
from jax.experimental import pallas as pl
from jax.experimental.pallas import tpu as pltpu

_TOPK = 200
_NUM_PERS = 4
_SUB = 40


def _fused_kernel(x_ref, pf_ref, w_ref, out_ref, pn_ref):
    @pl.when(pl.program_id(0) == 0)
    def _():
        pf = pf_ref[...]
        w = w_ref[...]
        for k in range(_NUM_PERS):
            ps = pf * w[k][None, :]
            pn_ref[k, :, :] = ps / jnp.maximum(
                jnp.sqrt(jnp.sum(ps * ps, axis=1, keepdims=True)), 1e-8)

    x = x_ref[...]
    w = w_ref[...]
    acc = jnp.zeros((x.shape[0], pn_ref.shape[1]), jnp.float32)
    for k in range(_NUM_PERS):
        xs = x * w[k][None, :]
        d4 = jnp.maximum(
            jnp.sqrt(jnp.sum(xs * xs, axis=1, keepdims=True)),
            1e-8) * jnp.float32(_NUM_PERS)
        acc = acc + jax.lax.dot_general(
            xs / d4, pn_ref[k, :, :], (((1,), (1,)), ((), ())),
            preferred_element_type=jnp.float32)
    att = acc

    u = jax.lax.bitcast_convert_type(att, jnp.int32)
    key = u ^ ((u >> 31) & jnp.int32(0x7FFFFFFF))

    kf = jnp.float32(_TOPK)
    for t in range(att.shape[0] // _SUB):
        sl = slice(t * _SUB, (t + 1) * _SUB)
        mk = key[sl, :]
        thr = jnp.full((_SUB, 1), jnp.int32(-(1 << 30)))
        for bit in range(30, -1, -1):
            cand = thr + jnp.int32(1 << bit)
            m = jnp.where(mk >= cand, 1.0, 0.0)
            cnt = jnp.sum(m, axis=1, keepdims=True)
            thr = jnp.where(cnt >= kf, cand, thr)
        out_ref[sl, :] = jnp.where(mk >= thr, att[sl, :], 0.0)


def kernel(node_features, pivot_features, weight_tensor):
    n, d = node_features.shape
    p, _ = pivot_features.shape
    block_n = 400
    assert n % block_n == 0
    w_pad = jnp.zeros((8, d), weight_tensor.dtype).at[:_NUM_PERS].set(
        weight_tensor)
    return pl.pallas_call(
        _fused_kernel,
        grid=(n // block_n,),
        in_specs=[
            pl.BlockSpec((block_n, d), lambda i: (i, 0)),
            pl.BlockSpec((p, d), lambda i: (0, 0)),
            pl.BlockSpec((8, d), lambda i: (0, 0)),
        ],
        out_specs=pl.BlockSpec((block_n, p), lambda i: (i, 0)),
        out_shape=jax.ShapeDtypeStruct((n, p), jnp.float32),
        scratch_shapes=[pltpu.VMEM((_NUM_PERS, p, d), jnp.float32)],
    )(node_features, pivot_features, w_pad)
